# baseline (speedup 1.0000x reference)
import functools

import jax
import jax.numpy as jnp
from jax import lax
from jax.experimental import pallas as pl
from jax.experimental.pallas import tpu as pltpu
from jax.experimental.pallas import tpu_sc as plsc

NUM_TOKENS = 8192
NUM_CODES = 8192
DIM = 32
TOK_BLK = 1024
CB_BLK = 2048
T_TILES = NUM_TOKENS // TOK_BLK
C_TILES = NUM_CODES // CB_BLK


def _tc_body(xt_ref, e_ref, idx_ref, loss_ref):
    t = pl.program_id(0)
    xt = xt_ref[...]
    xb2 = (xt + xt).astype(jnp.bfloat16)
    in_row = jnp.sum(xt * xt, axis=0, keepdims=True)
    inf_f = jnp.float32(jnp.inf)

    best = None
    true = None
    idxf = None
    for c in range(C_TILES):
        e = e_ref[pl.ds(c * CB_BLK, CB_BLK), :]
        eb = e.astype(jnp.bfloat16)
        mm2 = lax.dot_general(eb, xb2, (((1,), (0,)), ((), ())),
                              preferred_element_type=jnp.float32)
        en_col = jnp.sum(e * e, axis=1, keepdims=True)
        d2 = (in_row + en_col) - mm2
        dd = jnp.sqrt(jnp.maximum(d2, 0.0))
        dmin = jnp.min(dd, axis=0, keepdims=True)
        iota_col = lax.broadcasted_iota(
            jnp.int32, (CB_BLK, 1), 0).astype(jnp.float32)
        cand = jnp.where(dd == dmin, iota_col, inf_f)
        lidx = jnp.min(cand, axis=0, keepdims=True) + (c * CB_BLK)
        dmin_bf = dmin.astype(jnp.bfloat16).astype(jnp.float32)
        if c == 0:
            best, true, idxf = dmin_bf, dmin, lidx
        else:
            better = dmin < best
            idxf = jnp.where(better, lidx, idxf)
            best = jnp.where(better, dmin_bf, best)
            true = jnp.minimum(true, dmin)

    idx_ref[...] = idxf.astype(jnp.int32).reshape(1, 1, TOK_BLK)
    part = jnp.sum(true * true)
    acc = jnp.where(t == 0, jnp.zeros((1, 1), jnp.float32), loss_ref[...])
    tot = acc + part
    inv_n = 1.0 / float(NUM_TOKENS * DIM)
    loss_ref[...] = jnp.where(t == T_TILES - 1, tot * inv_n, tot)


def _tc_argmin(flat_t, embeddings, interpret=False):
    return pl.pallas_call(
        _tc_body,
        grid=(T_TILES,),
        in_specs=[
            pl.BlockSpec((DIM, TOK_BLK), lambda t: (0, t)),
            pl.BlockSpec((NUM_CODES, DIM), lambda t: (0, 0)),
        ],
        out_specs=[
            pl.BlockSpec((1, 1, TOK_BLK), lambda t: (t, 0, 0)),
            pl.BlockSpec((1, 1), lambda t: (0, 0)),
        ],
        out_shape=[
            jax.ShapeDtypeStruct((T_TILES, 1, TOK_BLK), jnp.int32),
            jax.ShapeDtypeStruct((1, 1), jnp.float32),
        ],
        interpret=interpret,
    )(flat_t, embeddings)


def _sc_gather_st(embeddings, indices, flat):
    info = plsc.get_sparse_core_info()
    nw = info.num_cores * info.num_subcores
    b_per_w = NUM_TOKENS // nw
    chunk = 128
    n_chunks = b_per_w // chunk
    mesh = plsc.VectorSubcoreMesh(core_axis_name="c", subcore_axis_name="s")

    @functools.partial(
        pl.kernel,
        mesh=mesh,
        out_type=jax.ShapeDtypeStruct((NUM_TOKENS, DIM), jnp.float32),
        scratch_types=[
            pltpu.VMEM((n_chunks, chunk), jnp.int32),
            pltpu.VMEM((n_chunks, chunk, DIM), jnp.float32),
            pltpu.VMEM((n_chunks, chunk, DIM), jnp.float32),
            pltpu.SemaphoreType.DMA,
        ],
        compiler_params=pltpu.CompilerParams(use_tc_tiling_on_sc=False),
    )
    def k(table_hbm, idx_hbm, z_hbm, out_hbm, idx_v, rows_v, z_v, sem):
        wid = lax.axis_index("s") * info.num_cores + lax.axis_index("c")
        base = wid * b_per_w
        for b in range(n_chunks):
            pltpu.sync_copy(idx_hbm.at[pl.ds(base + b * chunk, chunk)],
                            idx_v.at[b])
            pltpu.async_copy(table_hbm.at[idx_v.at[b]], rows_v.at[b],
                             sem).wait()
            pltpu.sync_copy(z_hbm.at[pl.ds(base + b * chunk, chunk)],
                            z_v.at[b])

            def st_row(i, carry):
                for h in range(DIM // 16):
                    sl = pl.ds(h * 16, 16)
                    q = rows_v[b, i, sl]
                    z = z_v[b, i, sl]
                    rows_v[b, i, sl] = z + (q - z)
                return carry

            lax.fori_loop(0, chunk, st_row, 0)
            pltpu.sync_copy(rows_v.at[b],
                            out_hbm.at[pl.ds(base + b * chunk, chunk)])

    return k(embeddings, indices, flat)


def kernel(inputs, embeddings):
    flat = inputs.reshape(NUM_TOKENS, DIM)
    idx3, loss_raw = _tc_argmin(flat.T, embeddings)
    indices = idx3.reshape(NUM_TOKENS)
    qst_flat = _sc_gather_st(embeddings, indices, flat)
    quantized_st = qst_flat.reshape(inputs.shape)
    loss = loss_raw[0, 0]
    return (quantized_st, loss, indices)


# speedup vs baseline: 1.0260x; 1.0260x over previous
import functools

import jax
import jax.numpy as jnp
from jax import lax
from jax.experimental import pallas as pl
from jax.experimental.pallas import tpu as pltpu
from jax.experimental.pallas import tpu_sc as plsc

NUM_TOKENS = 8192
NUM_CODES = 8192
DIM = 32
TOK_BLK = 1024
CB_BLK = 2048
T_TILES = NUM_TOKENS // TOK_BLK
C_TILES = NUM_CODES // CB_BLK


def _tc_body(xt_ref, e_ref, idx_ref, loss_ref):
    t = pl.program_id(0)
    xt = xt_ref[...]
    xb2 = (xt + xt).astype(jnp.bfloat16)
    in_row = jnp.sum(xt * xt, axis=0, keepdims=True)
    inf_f = jnp.float32(jnp.inf)

    best = None
    true = None
    idxf = None
    for c in range(C_TILES):
        e = e_ref[pl.ds(c * CB_BLK, CB_BLK), :]
        eb = e.astype(jnp.bfloat16)
        mm2 = lax.dot_general(eb, xb2, (((1,), (0,)), ((), ())),
                              preferred_element_type=jnp.float32)
        en_col = jnp.sum(e * e, axis=1, keepdims=True)
        d2 = (in_row + en_col) - mm2
        dd = jnp.sqrt(jnp.maximum(d2, 0.0))
        dmin = jnp.min(dd, axis=0, keepdims=True)
        iota_col = lax.broadcasted_iota(
            jnp.int32, (CB_BLK, 1), 0).astype(jnp.float32)
        cand = jnp.where(dd == dmin, iota_col, inf_f)
        lidx = jnp.min(cand, axis=0, keepdims=True) + (c * CB_BLK)
        dmin_bf = dmin.astype(jnp.bfloat16).astype(jnp.float32)
        if c == 0:
            best, true, idxf = dmin_bf, dmin, lidx
        else:
            better = dmin < best
            idxf = jnp.where(better, lidx, idxf)
            best = jnp.where(better, dmin_bf, best)
            true = jnp.minimum(true, dmin)

    idx_ref[...] = idxf.astype(jnp.int32).reshape(1, 1, TOK_BLK)
    part = jnp.sum(true * true)
    acc = jnp.where(t == 0, jnp.zeros((1, 1), jnp.float32), loss_ref[...])
    tot = acc + part
    inv_n = 1.0 / float(NUM_TOKENS * DIM)
    loss_ref[...] = jnp.where(t == T_TILES - 1, tot * inv_n, tot)


def _tc_argmin(flat_t, embeddings, interpret=False):
    return pl.pallas_call(
        _tc_body,
        grid=(T_TILES,),
        in_specs=[
            pl.BlockSpec((DIM, TOK_BLK), lambda t: (0, t)),
            pl.BlockSpec((NUM_CODES, DIM), lambda t: (0, 0)),
        ],
        out_specs=[
            pl.BlockSpec((1, 1, TOK_BLK), lambda t: (t, 0, 0)),
            pl.BlockSpec((1, 1), lambda t: (0, 0)),
        ],
        out_shape=[
            jax.ShapeDtypeStruct((T_TILES, 1, TOK_BLK), jnp.int32),
            jax.ShapeDtypeStruct((1, 1), jnp.float32),
        ],
        interpret=interpret,
    )(flat_t, embeddings)


def _sc_gather(embeddings, indices):
    info = plsc.get_sparse_core_info()
    nw = info.num_cores * info.num_subcores
    b_per_w = NUM_TOKENS // nw
    chunk = 128
    n_chunks = b_per_w // chunk
    mesh = plsc.VectorSubcoreMesh(core_axis_name="c", subcore_axis_name="s")

    @functools.partial(
        pl.kernel,
        mesh=mesh,
        out_type=jax.ShapeDtypeStruct((NUM_TOKENS, DIM), jnp.float32),
        scratch_types=[
            pltpu.VMEM((n_chunks, chunk), jnp.int32),
            pltpu.VMEM((n_chunks, chunk, DIM), jnp.float32),
            pltpu.SemaphoreType.DMA,
        ],
        compiler_params=pltpu.CompilerParams(use_tc_tiling_on_sc=False),
    )
    def k(table_hbm, idx_hbm, out_hbm, idx_v, rows_v, sem):
        wid = lax.axis_index("s") * info.num_cores + lax.axis_index("c")
        base = wid * b_per_w
        for b in range(n_chunks):
            pltpu.sync_copy(idx_hbm.at[pl.ds(base + b * chunk, chunk)],
                            idx_v.at[b])
            pltpu.async_copy(table_hbm.at[idx_v.at[b]], rows_v.at[b],
                             sem).wait()
            pltpu.sync_copy(rows_v.at[b],
                            out_hbm.at[pl.ds(base + b * chunk, chunk)])

    return k(embeddings, indices)


def kernel(inputs, embeddings):
    flat = inputs.reshape(NUM_TOKENS, DIM)
    idx3, loss_raw = _tc_argmin(flat.T, embeddings)
    indices = idx3.reshape(NUM_TOKENS)
    q_flat = _sc_gather(embeddings, indices)
    quantized = q_flat.reshape(inputs.shape)
    loss = loss_raw[0, 0]
    quantized_st = inputs + lax.stop_gradient(quantized - inputs)
    return (quantized_st, loss, indices)
